# baseline (speedup 1.0000x reference)
import functools

import jax
import jax.numpy as jnp
from jax import lax
from jax.experimental import pallas as pl
from jax.experimental.pallas import tpu as pltpu
from jax.experimental.pallas import tpu_sc as plsc

NC = 2
NS = 16
NW = NC * NS
L = 16

CHUNK = 128


def _bias_kernel(b_per_w, n_chunks):
    mesh = plsc.VectorSubcoreMesh(core_axis_name="c", subcore_axis_name="s")
    B = b_per_w * NW

    @functools.partial(
        pl.kernel,
        mesh=mesh,
        compiler_params=pltpu.CompilerParams(
            needs_layout_passes=False, use_tc_tiling_on_sc=False
        ),
        out_type=jax.ShapeDtypeStruct((B,), jnp.float32),
        scratch_types=[
            pltpu.VMEM((CHUNK,), jnp.int32),
            pltpu.VMEM((CHUNK,), jnp.int32),
            pltpu.VMEM((2, CHUNK, 1), jnp.float32),
            pltpu.VMEM((2, CHUNK, 1), jnp.float32),
            pltpu.VMEM((b_per_w,), jnp.float32),
            pltpu.SemaphoreType.DMA,
        ],
    )
    def k(uraw_hbm, iraw_hbm, ub_hbm, ib_hbm, out_hbm,
          uraw_v, iraw_v, ubs_v, ibs_v, out_v, sem):
        wid = lax.axis_index("s") * NC + lax.axis_index("c")
        base = wid * b_per_w

        def fire(j, buf):
            csl = pl.ds(base + j * CHUNK, CHUNK)
            pltpu.sync_copy(uraw_hbm.at[csl], uraw_v)
            pltpu.sync_copy(iraw_hbm.at[csl], iraw_v)
            return [
                pltpu.async_copy(ub_hbm.at[uraw_v], ubs_v.at[buf], sem),
                pltpu.async_copy(ib_hbm.at[iraw_v], ibs_v.at[buf], sem),
            ]

        pending = fire(0, 0)
        for j in range(n_chunks):
            buf = j % 2
            for c in pending:
                c.wait()
            if j + 1 < n_chunks:
                pending = fire(j + 1, 1 - buf)

            def body(g, _, _j=j, _buf=buf):
                gsl = pl.ds(g * L, L)
                rows = g * L + lax.broadcasted_iota(jnp.int32, (L,), 0)
                zero = jnp.zeros((L,), jnp.int32)
                s = (plsc.load_gather(ubs_v, [jnp.full((L,), _buf), rows, zero])
                     + plsc.load_gather(ibs_v, [jnp.full((L,), _buf), rows, zero]))
                out_v[pl.ds(_j * CHUNK + g * L, L)] = s
                return _

            lax.fori_loop(0, CHUNK // L, body, None)

        pltpu.sync_copy(out_v, out_hbm.at[pl.ds(base, b_per_w)])

    return k


def _dot_kernel(b_per_w, n_feats):
    mesh = plsc.VectorSubcoreMesh(core_axis_name="c", subcore_axis_name="s")
    B = b_per_w * NW

    @functools.partial(
        pl.kernel,
        mesh=mesh,
        compiler_params=pltpu.CompilerParams(needs_layout_passes=False),
        out_type=jax.ShapeDtypeStruct((B,), jnp.float32),
        scratch_types=[
            pltpu.VMEM((n_feats, b_per_w), jnp.float32),
            pltpu.VMEM((n_feats, b_per_w), jnp.float32),
            pltpu.VMEM((b_per_w,), jnp.float32),
            pltpu.VMEM((b_per_w,), jnp.float32),
            pltpu.SemaphoreType.DMA,
        ],
    )
    def k(uvt_hbm, ivt_hbm, bias_hbm, out_hbm, us_v, is_v, b_v, out_v, sem):
        wid = lax.axis_index("s") * NC + lax.axis_index("c")
        base = wid * b_per_w
        wsl = pl.ds(base, b_per_w)

        cps = [
            pltpu.async_copy(uvt_hbm.at[:, wsl], us_v, sem),
            pltpu.async_copy(ivt_hbm.at[:, wsl], is_v, sem),
            pltpu.async_copy(bias_hbm.at[wsl], b_v, sem),
        ]
        for c in cps:
            c.wait()

        def compute(g, _):
            gsl = pl.ds(g * L, L)
            acc = b_v[gsl]
            for f in range(n_feats):
                acc = acc + us_v[f, gsl] * is_v[f, gsl]
            out_v[gsl] = acc
            return _

        lax.fori_loop(0, b_per_w // L, compute, None)
        pltpu.sync_copy(out_v, out_hbm.at[wsl])

    return k


def kernel(user, item, u_bias, i_bias, u_embed, i_embed):
    B = user.shape[0]
    n_feats = u_embed.shape[1]
    b_per_w = B // NW

    user = user.astype(jnp.int32)
    item = item.astype(jnp.int32)

    uvt = jnp.take(u_embed, user, axis=0).T
    ivt = jnp.take(i_embed, item, axis=0).T

    bias = _bias_kernel(b_per_w, b_per_w // CHUNK)(user, item, u_bias, i_bias)
    return _dot_kernel(b_per_w, n_feats)(uvt, ivt, bias)
